# baseline (speedup 1.0000x reference)
import functools

import jax
import jax.numpy as jnp
import numpy as np
from jax import lax
from jax.experimental import pallas as pl
from jax.experimental.pallas import tpu as pltpu
from jax.experimental.pallas import tpu_sc as plsc

_NF = 100
_D = 128
_GW = _D // 2
_ROW_NP, _COL_NP = np.triu_indices(_NF, k=1)
_P = _ROW_NP.size

_BBLK = 128

_NW = 32
_CH = 4
_PPAD = 4960
_OUTPAD = _CH * _P + 24

_IDXR_NP = np.concatenate(
    [_ROW_NP >> 1, np.zeros(_PPAD - _P, np.int64)]).astype(np.int32)
_IDXC_NP = np.concatenate(
    [_COL_NP, np.zeros(_PPAD - _P, np.int64)]).astype(np.int32)
_SHA_NP = np.concatenate(
    [(_ROW_NP & 1) * 16, np.zeros(_PPAD - _P, np.int64)]).astype(np.int32)


def _gram_body(x_ref, g_ref):
    for b in range(_BBLK):
        xb = x_ref[b].astype(jnp.bfloat16)
        res = jax.lax.dot_general(
            xb, xb, (((1,), (1,)), ((), ())),
            preferred_element_type=jnp.float32)
        packed = pltpu.bitcast(res.astype(jnp.bfloat16), jnp.int32)
        g_ref[b * (_NF // 2):(b + 1) * (_NF // 2), 0:_NF] = packed


def _gram(x_embed):
    B = x_embed.shape[0]
    return pl.pallas_call(
        _gram_body,
        grid=(B // _BBLK,),
        compiler_params=pltpu.CompilerParams(
            dimension_semantics=("parallel",)),
        in_specs=[pl.BlockSpec((_BBLK, _NF, _D), lambda i: (i, 0, 0))],
        out_specs=pl.BlockSpec((_BBLK * (_NF // 2), _D), lambda i: (i, 0)),
        out_shape=jax.ShapeDtypeStruct((B * (_NF // 2), _D), jnp.int32),
    )(x_embed)


def _sc_gather(g_i32, idxr, idxc, sha, batch):
    chunks_per_w = batch // (_CH * _NW)
    mesh = plsc.VectorSubcoreMesh(core_axis_name="c", subcore_axis_name="s")

    @functools.partial(
        pl.kernel, mesh=mesh,
        compiler_params=pltpu.CompilerParams(needs_layout_passes=False),
        out_type=jax.ShapeDtypeStruct((batch * _P,), jnp.float32),
        scratch_types=[
            pltpu.VMEM((_PPAD,), jnp.int32),
            pltpu.VMEM((_PPAD,), jnp.int32),
            pltpu.VMEM((_PPAD,), jnp.int32),
            pltpu.VMEM((_CH * _NF // 2, _D), jnp.int32),
            pltpu.VMEM((_CH * _NF // 2, _D), jnp.int32),
            pltpu.VMEM((_OUTPAD,), jnp.float32),
            pltpu.VMEM((_OUTPAD,), jnp.float32),
            pltpu.SemaphoreType.DMA,
            pltpu.SemaphoreType.DMA,
            pltpu.SemaphoreType.DMA,
            pltpu.SemaphoreType.DMA,
        ],
    )
    def k(g_hbm, idxr_hbm, idxc_hbm, sha_hbm, out_hbm, idxr_v, idxc_v,
          sha_v, rows0, rows1, outs0, outs1, is0, is1, os0, os1):
        wid = lax.axis_index("s") * 2 + lax.axis_index("c")
        cbase = wid * chunks_per_w
        rows_bufs = (rows0, rows1)
        out_bufs = (outs0, outs1)
        isems = (is0, is1)
        osems = (os0, os1)
        pltpu.sync_copy(idxr_hbm, idxr_v)
        pltpu.sync_copy(idxc_hbm, idxc_v)
        pltpu.sync_copy(sha_hbm, sha_v)
        rlen = _CH * _NF // 2

        def in_copy(ci, b):
            return pltpu.make_async_copy(
                g_hbm.at[pl.ds((cbase + ci) * rlen, rlen)],
                rows_bufs[b], isems[b])

        def out_copy(ci, b):
            return pltpu.make_async_copy(
                out_bufs[b].at[pl.ds(0, _CH * _P)],
                out_hbm.at[pl.ds((cbase + ci) * (_CH * _P), _CH * _P)],
                osems[b])

        def gather_chunk(b):
            for r in range(_CH):
                roff = r * (_NF // 2)
                obase = r * _P

                @plsc.parallel_loop(0, _PPAD, step=16, unroll=8)
                def g_body(gi):
                    ir = idxr_v[pl.ds(gi, 16)] + roff
                    ic = idxc_v[pl.ds(gi, 16)]
                    pair = plsc.load_gather(rows_bufs[b], [ir, ic])
                    sh = sha_v[pl.ds(gi, 16)]
                    v = lax.shift_left(lax.shift_right_logical(pair, sh), 16)
                    out_bufs[b][pl.ds(obase + gi, 16)] = plsc.bitcast(
                        v, jnp.float32)

        in_copy(0, 0).start()
        in_copy(0, 0).wait()
        in_copy(1, 1).start()
        gather_chunk(0)
        out_copy(0, 0).start()
        in_copy(1, 1).wait()
        in_copy(2, 0).start()
        gather_chunk(1)
        out_copy(1, 1).start()

        def pair_body(t, carry):
            for sub in range(2):
                ci = 2 * t + sub
                in_copy(ci, sub).wait()
                in_copy(ci + 1, 1 - sub).start()
                out_copy(ci - 2, sub).wait()
                gather_chunk(sub)
                out_copy(ci, sub).start()
            return carry

        lax.fori_loop(1, chunks_per_w // 2 - 1, pair_body, 0)
        ci = chunks_per_w - 2
        in_copy(ci, 0).wait()
        in_copy(ci + 1, 1).start()
        out_copy(ci - 2, 0).wait()
        gather_chunk(0)
        out_copy(ci, 0).start()
        ci = chunks_per_w - 1
        in_copy(ci, 1).wait()
        out_copy(ci - 2, 1).wait()
        gather_chunk(1)
        out_copy(ci, 1).start()
        out_copy(chunks_per_w - 2, 0).wait()
        out_copy(chunks_per_w - 1, 1).wait()

    return k(g_i32, idxr, idxc, sha)


def kernel(x_embed):
    B = x_embed.shape[0]
    g = _gram(x_embed)
    idxr = jnp.asarray(_IDXR_NP)
    idxc = jnp.asarray(_IDXC_NP)
    sha = jnp.asarray(_SHA_NP)
    out = _sc_gather(g, idxr, idxc, sha, B)
    return out.reshape(B, _P)


# speedup vs baseline: 1.0026x; 1.0026x over previous
import functools

import jax
import jax.numpy as jnp
import numpy as np
from jax import lax
from jax.experimental import pallas as pl
from jax.experimental.pallas import tpu as pltpu
from jax.experimental.pallas import tpu_sc as plsc

_NF = 100
_D = 128
_GW = _D // 2
_ROW_NP, _COL_NP = np.triu_indices(_NF, k=1)
_P = _ROW_NP.size

_BBLK = 128

_NW = 32
_CH = 4
_PPAD = 4960
_OUTPAD = _CH * _P + 24

_IDXR_NP = np.concatenate(
    [_ROW_NP >> 1, np.zeros(_PPAD - _P, np.int64)]).astype(np.int32)
_IDXC_NP = np.concatenate(
    [_COL_NP, np.zeros(_PPAD - _P, np.int64)]).astype(np.int32)
_SHA_NP = np.concatenate(
    [(_ROW_NP & 1) * 16, np.zeros(_PPAD - _P, np.int64)]).astype(np.int32)


def _gram_body(x_ref, g_ref):
    for b in range(_BBLK):
        xb = x_ref[b].astype(jnp.bfloat16)
        res = jax.lax.dot_general(
            xb, xb, (((1,), (1,)), ((), ())),
            preferred_element_type=jnp.float32)
        packed = pltpu.bitcast(res.astype(jnp.bfloat16), jnp.int32)
        g_ref[b * (_NF // 2):(b + 1) * (_NF // 2), 0:_NF] = packed


def _gram(x_embed):
    B = x_embed.shape[0]
    nblk = B // _BBLK

    def run(x, k, nchunks):
        cblk = nblk // nchunks
        return pl.pallas_call(
            _gram_body,
            grid=(cblk,),
            compiler_params=pltpu.CompilerParams(
                dimension_semantics=("parallel",)),
            in_specs=[pl.BlockSpec(
                (_BBLK, _NF, _D), lambda i: (k * cblk + i, 0, 0))],
            out_specs=pl.BlockSpec(
                (_BBLK * (_NF // 2), _D), lambda i: (k * cblk + i, 0)),
            out_shape=jax.ShapeDtypeStruct(
                (B * (_NF // 2), _D), jnp.int32),
        )(x)

    return run


def _sc_gather(g_i32, idxr, idxc, sha, batch, chunk0=0):
    chunks_per_w = batch // (_CH * _NW)
    mesh = plsc.VectorSubcoreMesh(core_axis_name="c", subcore_axis_name="s")

    @functools.partial(
        pl.kernel, mesh=mesh,
        compiler_params=pltpu.CompilerParams(needs_layout_passes=False),
        out_type=jax.ShapeDtypeStruct((batch * _P,), jnp.float32),
        scratch_types=[
            pltpu.VMEM((_PPAD,), jnp.int32),
            pltpu.VMEM((_PPAD,), jnp.int32),
            pltpu.VMEM((_PPAD,), jnp.int32),
            pltpu.VMEM((_CH * _NF // 2, _D), jnp.int32),
            pltpu.VMEM((_CH * _NF // 2, _D), jnp.int32),
            pltpu.VMEM((_OUTPAD,), jnp.float32),
            pltpu.VMEM((_OUTPAD,), jnp.float32),
            pltpu.SemaphoreType.DMA,
            pltpu.SemaphoreType.DMA,
            pltpu.SemaphoreType.DMA,
            pltpu.SemaphoreType.DMA,
        ],
    )
    def k(g_hbm, idxr_hbm, idxc_hbm, sha_hbm, out_hbm, idxr_v, idxc_v,
          sha_v, rows0, rows1, outs0, outs1, is0, is1, os0, os1):
        wid = lax.axis_index("s") * 2 + lax.axis_index("c")
        cbase = wid * chunks_per_w
        gbase = chunk0 + cbase
        rows_bufs = (rows0, rows1)
        out_bufs = (outs0, outs1)
        isems = (is0, is1)
        osems = (os0, os1)
        pltpu.sync_copy(idxr_hbm, idxr_v)
        pltpu.sync_copy(idxc_hbm, idxc_v)
        pltpu.sync_copy(sha_hbm, sha_v)
        rlen = _CH * _NF // 2

        def in_copy(ci, b):
            return pltpu.make_async_copy(
                g_hbm.at[pl.ds((gbase + ci) * rlen, rlen)],
                rows_bufs[b], isems[b])

        def out_copy(ci, b):
            return pltpu.make_async_copy(
                out_bufs[b].at[pl.ds(0, _CH * _P)],
                out_hbm.at[pl.ds((cbase + ci) * (_CH * _P), _CH * _P)],
                osems[b])

        def gather_chunk(b):
            for r in range(_CH):
                roff = r * (_NF // 2)
                obase = r * _P

                @plsc.parallel_loop(0, _PPAD, step=16, unroll=8)
                def g_body(gi):
                    ir = idxr_v[pl.ds(gi, 16)] + roff
                    ic = idxc_v[pl.ds(gi, 16)]
                    pair = plsc.load_gather(rows_bufs[b], [ir, ic])
                    sh = sha_v[pl.ds(gi, 16)]
                    v = lax.shift_left(lax.shift_right_logical(pair, sh), 16)
                    out_bufs[b][pl.ds(obase + gi, 16)] = plsc.bitcast(
                        v, jnp.float32)

        in_copy(0, 0).start()
        in_copy(0, 0).wait()
        in_copy(1, 1).start()
        gather_chunk(0)
        out_copy(0, 0).start()
        in_copy(1, 1).wait()
        in_copy(2, 0).start()
        gather_chunk(1)
        out_copy(1, 1).start()

        def pair_body(t, carry):
            for sub in range(2):
                ci = 2 * t + sub
                in_copy(ci, sub).wait()
                in_copy(ci + 1, 1 - sub).start()
                out_copy(ci - 2, sub).wait()
                gather_chunk(sub)
                out_copy(ci, sub).start()
            return carry

        lax.fori_loop(1, chunks_per_w // 2 - 1, pair_body, 0)
        ci = chunks_per_w - 2
        in_copy(ci, 0).wait()
        in_copy(ci + 1, 1).start()
        out_copy(ci - 2, 0).wait()
        gather_chunk(0)
        out_copy(ci, 0).start()
        ci = chunks_per_w - 1
        in_copy(ci, 1).wait()
        out_copy(ci - 2, 1).wait()
        gather_chunk(1)
        out_copy(ci, 1).start()
        out_copy(chunks_per_w - 2, 0).wait()
        out_copy(chunks_per_w - 1, 1).wait()

    return k(g_i32, idxr, idxc, sha)


def kernel(x_embed):
    B = x_embed.shape[0]
    idxr = jnp.asarray(_IDXR_NP)
    idxc = jnp.asarray(_IDXC_NP)
    sha = jnp.asarray(_SHA_NP)
    run = _gram(x_embed)
    K = 2
    CB = B // K
    outs = []
    for k in range(K):
        gk = run(x_embed, k, K)
        outs.append(_sc_gather(gk, idxr, idxc, sha, CB,
                               chunk0=k * (CB // _CH)))
    out = jnp.concatenate(outs)
    return out.reshape(B, _P)
